# baseline (speedup 1.0000x reference)
import functools

import jax
import jax.numpy as jnp
from jax import lax
from jax.experimental import pallas as pl
from jax.experimental.pallas import tpu as pltpu
from jax.experimental.pallas import tpu_sc as plsc

_NC = 2
_NS = 16
_NW = _NC * _NS

_WDIMS = 32
_CHUNK = 400
_NBUF = 8
_DIST = _NBUF // 2


def _make_lookup(n_idx: int):
  per_w = n_idx // _NW
  n_chunks = per_w // _CHUNK
  n_groups = n_chunks // _NBUF
  assert per_w * _NW == n_idx and n_chunks * _CHUNK == per_w
  assert n_groups * _NBUF == n_chunks and n_chunks >= _NBUF

  mesh = plsc.VectorSubcoreMesh(core_axis_name="c", subcore_axis_name="s")

  @functools.partial(
      pl.kernel,
      mesh=mesh,
      out_type=jax.ShapeDtypeStruct((n_idx, _WDIMS), jnp.float32),
      scratch_types=[
          pltpu.VMEM((per_w,), jnp.int32),
          pltpu.VMEM((_NBUF, _CHUNK, _WDIMS), jnp.float32),
          pltpu.SemaphoreType.DMA((_NBUF,)),
          pltpu.SemaphoreType.DMA((_NBUF,)),
      ],
      compiler_params=pltpu.CompilerParams(use_tc_tiling_on_sc=False),
  )
  def lookup(idx_hbm, table_hbm, out_hbm, idx_v, rows_v, gsem, osem):
    wid = lax.axis_index("s") * _NC + lax.axis_index("c")
    base = wid * per_w
    pltpu.sync_copy(idx_hbm.at[pl.ds(base, per_w)], idx_v)

    def gather_desc(j, buf):
      return pltpu.make_async_copy(
          table_hbm.at[idx_v.at[pl.ds(j * _CHUNK, _CHUNK)]],
          rows_v.at[buf],
          gsem.at[buf],
      )

    def out_desc(j, buf):
      return pltpu.make_async_copy(
          rows_v.at[buf],
          out_hbm.at[pl.ds(base + j * _CHUNK, _CHUNK)],
          osem.at[buf],
      )

    for p in range(_DIST):
      gather_desc(p, p).start()

    def step(j, b):
      @pl.when(j >= _DIST)
      def _():
        out_desc(j - _DIST, (b + _DIST) % _NBUF).wait()

      @pl.when(j + _DIST < n_chunks)
      def _():
        gather_desc(j + _DIST, (b + _DIST) % _NBUF).start()

      gather_desc(j, b).wait()
      out_desc(j, b).start()

    def group_body(g, carry):
      j0 = g * _NBUF
      for b in range(_NBUF):
        step(j0 + b, b)
      return carry

    lax.fori_loop(0, n_groups, group_body, 0)
    for p in range(_DIST):
      j = n_chunks - _DIST + p
      out_desc(j, j % _NBUF).wait()

  return lookup


def kernel(inputs, table):
  batch, hist = inputs.shape
  n_idx = batch * hist
  half = n_idx // 2
  idx = inputs.T.reshape(n_idx).astype(jnp.int32)
  lookup = _make_lookup(half)
  o1 = lookup(idx[:half], table)
  o2 = lookup(idx[half:], table)
  out = jnp.concatenate([o1, o2], axis=0)
  return out.reshape(hist, batch, table.shape[1]).transpose(1, 0, 2)


# speedup vs baseline: 1.2931x; 1.2931x over previous
import functools

import jax
import jax.numpy as jnp
from jax import lax
from jax.experimental import pallas as pl
from jax.experimental.pallas import tpu as pltpu
from jax.experimental.pallas import tpu_sc as plsc

_NC = 2
_NS = 16
_NW = _NC * _NS

_WDIMS = 32
_CHUNK = 400
_NBUF = 8
_DIST = _NBUF // 2


def _make_lookup(n_idx: int):
  per_w = n_idx // _NW
  n_chunks = per_w // _CHUNK
  n_groups = n_chunks // _NBUF
  assert per_w * _NW == n_idx and n_chunks * _CHUNK == per_w
  assert n_groups * _NBUF == n_chunks and n_chunks >= _NBUF

  mesh = plsc.VectorSubcoreMesh(core_axis_name="c", subcore_axis_name="s")

  @functools.partial(
      pl.kernel,
      mesh=mesh,
      out_type=jax.ShapeDtypeStruct((n_idx, _WDIMS), jnp.float32),
      scratch_types=[
          pltpu.VMEM((per_w,), jnp.int32),
          pltpu.VMEM((_NBUF, _CHUNK, _WDIMS), jnp.float32),
          pltpu.SemaphoreType.DMA((_NBUF,)),
          pltpu.SemaphoreType.DMA((_NBUF,)),
      ],
      compiler_params=pltpu.CompilerParams(use_tc_tiling_on_sc=False),
  )
  def lookup(idx_hbm, table_hbm, out_hbm, idx_v, rows_v, gsem, osem):
    wid = lax.axis_index("s") * _NC + lax.axis_index("c")
    base = wid * per_w
    pltpu.sync_copy(idx_hbm.at[pl.ds(base, per_w)], idx_v)

    def gather_desc(j, buf):
      return pltpu.make_async_copy(
          table_hbm.at[idx_v.at[pl.ds(j * _CHUNK, _CHUNK)]],
          rows_v.at[buf],
          gsem.at[buf],
      )

    def out_desc(j, buf):
      return pltpu.make_async_copy(
          rows_v.at[buf],
          out_hbm.at[pl.ds(base + j * _CHUNK, _CHUNK)],
          osem.at[buf],
      )

    for p in range(_DIST):
      gather_desc(p, p).start()

    def step(j, b):
      @pl.when(j >= _DIST)
      def _():
        out_desc(j - _DIST, (b + _DIST) % _NBUF).wait()

      @pl.when(j + _DIST < n_chunks)
      def _():
        gather_desc(j + _DIST, (b + _DIST) % _NBUF).start()

      gather_desc(j, b).wait()
      out_desc(j, b).start()

    def group_body(g, carry):
      j0 = g * _NBUF
      for b in range(_NBUF):
        step(j0 + b, b)
      return carry

    lax.fori_loop(0, n_groups, group_body, 0)
    for p in range(_DIST):
      j = n_chunks - _DIST + p
      out_desc(j, j % _NBUF).wait()

  return lookup


def kernel(inputs, table):
  batch, hist = inputs.shape
  n_idx = batch * hist
  idx = inputs.T.reshape(n_idx).astype(jnp.int32)
  out = _make_lookup(n_idx)(idx, table)
  return out.reshape(hist, batch, table.shape[1]).transpose(1, 0, 2)
